# Initial kernel scaffold; baseline (speedup 1.0000x reference)
import jax
import jax.numpy as jnp
from jax.experimental import pallas as pl


def kernel(x, weight):
    raise NotImplementedError("write your pallas kernel here")


# speedup vs baseline: 1.8190x; 1.8190x over previous
import functools

import jax
import jax.numpy as jnp
from jax.experimental import pallas as pl

HIDDEN = 2048
NUM_EXPERTS = 64
TOKENS = 16384
BT = 1024


def _router_block(x_ref, w_ref, wout_ref, iout_ref):
    logits = jax.lax.dot_general(
        x_ref[...], w_ref[...],
        dimension_numbers=(((1,), (1,)), ((), ())),
        preferred_element_type=jnp.float32,
    )
    idx = jax.lax.broadcasted_iota(jnp.int32, logits.shape, 1)
    m0 = jnp.max(logits, axis=-1, keepdims=True)
    i0 = jnp.min(jnp.where(logits == m0, idx, NUM_EXPERTS), axis=-1,
                 keepdims=True)
    masked = jnp.where(idx == i0, -jnp.inf, logits)
    m1 = jnp.max(masked, axis=-1, keepdims=True)
    i1 = jnp.min(jnp.where(masked == m1, idx, NUM_EXPERTS), axis=-1,
                 keepdims=True)
    e1 = jnp.exp(m1 - m0)
    denom = 1.0 + e1
    w0 = 1.0 / denom
    w1 = e1 / denom
    wout_ref[...] = jnp.concatenate([w0, w1], axis=-1)
    iout_ref[...] = jnp.concatenate([i0, i1], axis=-1)


@jax.jit
def kernel(x, weight):
    grid = (TOKENS // BT,)
    weights, experts = pl.pallas_call(
        _router_block,
        grid=grid,
        in_specs=[
            pl.BlockSpec((BT, HIDDEN), lambda i: (i, 0)),
            pl.BlockSpec((NUM_EXPERTS, HIDDEN), lambda i: (0, 0)),
        ],
        out_specs=[
            pl.BlockSpec((BT, 2), lambda i: (i, 0)),
            pl.BlockSpec((BT, 2), lambda i: (i, 0)),
        ],
        out_shape=[
            jax.ShapeDtypeStruct((TOKENS, 2), jnp.float32),
            jax.ShapeDtypeStruct((TOKENS, 2), jnp.int32),
        ],
    )(x, weight)
    return (weights, experts)


# speedup vs baseline: 1.8906x; 1.0393x over previous
import functools

import jax
import jax.numpy as jnp
from jax.experimental import pallas as pl

HIDDEN = 2048
NUM_EXPERTS = 64
TOKENS = 16384
BT = 2048


def _router_block(x_ref, w_ref, wout_ref, iout_ref):
    logits = jax.lax.dot_general(
        x_ref[...], w_ref[...],
        dimension_numbers=(((1,), (1,)), ((), ())),
        preferred_element_type=jnp.float32,
    )
    idx = jax.lax.broadcasted_iota(jnp.int32, logits.shape, 1)
    m0 = jnp.max(logits, axis=-1, keepdims=True)
    i0 = jnp.min(jnp.where(logits == m0, idx, NUM_EXPERTS), axis=-1,
                 keepdims=True)
    masked = jnp.where(idx == i0, -jnp.inf, logits)
    m1 = jnp.max(masked, axis=-1, keepdims=True)
    i1 = jnp.min(jnp.where(masked == m1, idx, NUM_EXPERTS), axis=-1,
                 keepdims=True)
    e1 = jnp.exp(m1 - m0)
    denom = 1.0 + e1
    w0 = 1.0 / denom
    w1 = e1 / denom
    wout_ref[...] = jnp.concatenate([w0, w1], axis=-1)
    iout_ref[...] = jnp.concatenate([i0, i1], axis=-1)


@jax.jit
def kernel(x, weight):
    grid = (TOKENS // BT,)
    weights, experts = pl.pallas_call(
        _router_block,
        grid=grid,
        in_specs=[
            pl.BlockSpec((BT, HIDDEN), lambda i: (i, 0)),
            pl.BlockSpec((NUM_EXPERTS, HIDDEN), lambda i: (0, 0)),
        ],
        out_specs=[
            pl.BlockSpec((BT, 2), lambda i: (i, 0)),
            pl.BlockSpec((BT, 2), lambda i: (i, 0)),
        ],
        out_shape=[
            jax.ShapeDtypeStruct((TOKENS, 2), jnp.float32),
            jax.ShapeDtypeStruct((TOKENS, 2), jnp.int32),
        ],
    )(x, weight)
    return (weights, experts)
